# Initial kernel scaffold; baseline (speedup 1.0000x reference)
import jax
import jax.numpy as jnp
from jax.experimental import pallas as pl


def kernel(xyz, points, new_xyz, new_points, affine_alpha, affine_beta):
    raise NotImplementedError("write your pallas kernel here")


# speedup vs baseline: 1.1899x; 1.1899x over previous
import functools

import jax
import jax.numpy as jnp
from jax import lax
from jax.experimental import pallas as pl
from jax.experimental.pallas import tpu as pltpu
from jax.experimental.pallas import tpu_sc as plsc

KNN = 32
B, N, S = 8, 8192, 512
CP = 256
CX = 5
CXP = 16
CC = CP + CX
COUT = CC + CP
NW = 32
GATHER_CHUNK = 128
SB = 128
NORM_N = S * KNN * CC


def _sc_gather(points_flat, xyz_flat, idx_flat):
    ni = idx_flat.shape[0]
    per_w = ni // NW
    n_chunks = per_w // GATHER_CHUNK
    mesh = plsc.VectorSubcoreMesh(core_axis_name="c", subcore_axis_name="s")

    @functools.partial(
        pl.kernel,
        mesh=mesh,
        compiler_params=pltpu.CompilerParams(use_tc_tiling_on_sc=False),
        out_type=(
            jax.ShapeDtypeStruct((ni, CP), jnp.float32),
            jax.ShapeDtypeStruct((ni, CXP), jnp.float32),
        ),
        scratch_types=[
            pltpu.VMEM((GATHER_CHUNK,), jnp.int32),
            pltpu.VMEM((GATHER_CHUNK, CP), jnp.float32),
            pltpu.VMEM((GATHER_CHUNK, CXP), jnp.float32),
            pltpu.SemaphoreType.DMA,
            pltpu.SemaphoreType.DMA,
        ],
    )
    def k(pts_hbm, xyz_hbm, idx_hbm, op_hbm, ox_hbm, idx_v, rp_v, rx_v, s1, s2):
        wid = lax.axis_index("s") * 2 + lax.axis_index("c")
        base = wid * per_w

        @pl.loop(0, n_chunks)
        def _(ci):
            off = base + ci * GATHER_CHUNK
            pltpu.sync_copy(idx_hbm.at[pl.ds(off, GATHER_CHUNK)], idx_v)
            c1 = pltpu.async_copy(pts_hbm.at[idx_v], rp_v, s1)
            c2 = pltpu.async_copy(xyz_hbm.at[idx_v], rx_v, s2)
            c1.wait()
            c2.wait()
            pltpu.sync_copy(rp_v, op_hbm.at[pl.ds(off, GATHER_CHUNK)])
            pltpu.sync_copy(rx_v, ox_hbm.at[pl.ds(off, GATHER_CHUNK)])

    return k(points_flat, xyz_flat, idx_flat)


def _stats_kernel(gp_ref, gx_ref, ss_ref):
    @pl.when(pl.program_id(1) == 0)
    def _():
        ss_ref[...] = jnp.zeros_like(ss_ref)

    g3 = gp_ref[0]
    x3 = gx_ref[0]
    mp = jnp.mean(g3, axis=1, keepdims=True)
    mx = jnp.mean(x3, axis=1, keepdims=True)
    cp = g3 - mp
    cx = x3 - mx
    tot = jnp.sum(cp * cp) + jnp.sum(cx * cx)
    ss_ref[...] += jnp.full((1, 1, 128), tot, jnp.float32)


def _stats(gp, gx):
    return pl.pallas_call(
        _stats_kernel,
        grid=(B, S // SB),
        in_specs=[
            pl.BlockSpec((1, SB, KNN, CP), lambda b, s: (b, s, 0, 0)),
            pl.BlockSpec((1, SB, KNN, CXP), lambda b, s: (b, s, 0, 0)),
        ],
        out_specs=pl.BlockSpec((1, 1, 128), lambda b, s: (b, 0, 0)),
        out_shape=jax.ShapeDtypeStruct((B, 1, 128), jnp.float32),
    )(gp, gx)


def _apply_kernel(gp_ref, gx_ref, ss_ref, np_ref, ap_ref, ax_ref,
                  bp_ref, bx_ref, out_ref):
    g3 = gp_ref[0]
    x3 = gx_ref[0]
    mp = jnp.mean(g3, axis=1, keepdims=True)
    mx = jnp.mean(x3, axis=1, keepdims=True)
    std = jnp.sqrt(ss_ref[0, 0, 0] / (NORM_N - 1))
    rstd = 1.0 / (std + 1e-5)
    outp = ap_ref[0][None, None, :] * ((g3 - mp) * rstd) + bp_ref[0][None, None, :]
    outx = ax_ref[0][None, None, :] * ((x3 - mx) * rstd) + bx_ref[0][None, None, :]
    npb = jnp.broadcast_to(np_ref[0][:, None, :], (SB, KNN, CP))
    out_ref[0, :, :, 0:CP] = outp
    out_ref[0, :, :, CP:CC] = outx[:, :, 0:CX]
    out_ref[0, :, :, CC:COUT] = npb


def _apply(gp, gx, ss, new_points, ap, ax, bp, bx):
    return pl.pallas_call(
        _apply_kernel,
        grid=(B, S // SB),
        in_specs=[
            pl.BlockSpec((1, SB, KNN, CP), lambda b, s: (b, s, 0, 0)),
            pl.BlockSpec((1, SB, KNN, CXP), lambda b, s: (b, s, 0, 0)),
            pl.BlockSpec((1, 1, 128), lambda b, s: (b, 0, 0)),
            pl.BlockSpec((1, SB, CP), lambda b, s: (b, s, 0)),
            pl.BlockSpec((1, CP), lambda b, s: (0, 0)),
            pl.BlockSpec((1, CXP), lambda b, s: (0, 0)),
            pl.BlockSpec((1, CP), lambda b, s: (0, 0)),
            pl.BlockSpec((1, CXP), lambda b, s: (0, 0)),
        ],
        out_specs=pl.BlockSpec((1, SB, KNN, COUT), lambda b, s: (b, s, 0, 0)),
        out_shape=jax.ShapeDtypeStruct((B, S, KNN, COUT), jnp.float32),
    )(gp, gx, ss, new_points, ap, ax, bp, bx)


def kernel(xyz, points, new_xyz, new_points, affine_alpha, affine_beta):
    dist = -2.0 * jnp.matmul(new_xyz, jnp.swapaxes(xyz, 1, 2))
    dist = dist + jnp.sum(new_xyz ** 2, axis=-1)[:, :, None]
    dist = dist + jnp.sum(xyz ** 2, axis=-1)[:, None, :]
    _, idx = jax.lax.top_k(-dist, KNN)

    idx_flat = (idx + (jnp.arange(B, dtype=jnp.int32) * N)[:, None, None])
    idx_flat = idx_flat.reshape(-1).astype(jnp.int32)
    points_flat = points.reshape(B * N, CP)
    xyz_flat = jnp.pad(xyz, ((0, 0), (0, 0), (0, CXP - CX))).reshape(B * N, CXP)

    gp_flat, gx_flat = _sc_gather(points_flat, xyz_flat, idx_flat)
    gp = gp_flat.reshape(B, S, KNN, CP)
    gx = gx_flat.reshape(B, S, KNN, CXP)

    ss = _stats(gp, gx)
    a = affine_alpha.reshape(CC)
    bta = affine_beta.reshape(CC)
    ap = a[:CP].reshape(1, CP)
    ax = jnp.pad(a[CP:], (0, CXP - CX)).reshape(1, CXP)
    bp = bta[:CP].reshape(1, CP)
    bx = jnp.pad(bta[CP:], (0, CXP - CX)).reshape(1, CXP)
    out = _apply(gp, gx, ss, new_points, ap, ax, bp, bx)
    return (new_xyz, out)


# speedup vs baseline: 5.0162x; 4.2157x over previous
import functools

import jax
import jax.numpy as jnp
from jax import lax
from jax.experimental import pallas as pl
from jax.experimental.pallas import tpu as pltpu
from jax.experimental.pallas import tpu_sc as plsc

KNN = 32
B, N, S = 8, 8192, 512
CP = 256
CX = 5
CXP = 16
CC = CP + CX
COUT = CC + CP
NW = 32
GATHER_CHUNK = 128
SB = 128
NORM_N = S * KNN * CC


def _sc_gather(points_flat, xyz_flat, idx_flat):
    ni = idx_flat.shape[0]
    per_w = ni // NW
    n_chunks = per_w // GATHER_CHUNK
    mesh = plsc.VectorSubcoreMesh(core_axis_name="c", subcore_axis_name="s")

    @functools.partial(
        pl.kernel,
        mesh=mesh,
        compiler_params=pltpu.CompilerParams(use_tc_tiling_on_sc=False),
        out_type=(
            jax.ShapeDtypeStruct((ni, CP), jnp.float32),
            jax.ShapeDtypeStruct((ni, CXP), jnp.float32),
        ),
        scratch_types=[
            pltpu.VMEM((GATHER_CHUNK,), jnp.int32),
            pltpu.VMEM((GATHER_CHUNK, CP), jnp.float32),
            pltpu.VMEM((GATHER_CHUNK, CXP), jnp.float32),
            pltpu.SemaphoreType.DMA,
            pltpu.SemaphoreType.DMA,
        ],
    )
    def k(pts_hbm, xyz_hbm, idx_hbm, op_hbm, ox_hbm, idx_v, rp_v, rx_v, s1, s2):
        wid = lax.axis_index("s") * 2 + lax.axis_index("c")
        base = wid * per_w

        @pl.loop(0, n_chunks)
        def _(ci):
            off = base + ci * GATHER_CHUNK
            pltpu.sync_copy(idx_hbm.at[pl.ds(off, GATHER_CHUNK)], idx_v)
            c1 = pltpu.async_copy(pts_hbm.at[idx_v], rp_v, s1)
            c2 = pltpu.async_copy(xyz_hbm.at[idx_v], rx_v, s2)
            c1.wait()
            c2.wait()
            pltpu.sync_copy(rp_v, op_hbm.at[pl.ds(off, GATHER_CHUNK)])
            pltpu.sync_copy(rx_v, ox_hbm.at[pl.ds(off, GATHER_CHUNK)])

    return k(points_flat, xyz_flat, idx_flat)


def _stats_kernel(gp_ref, gx_ref, ss_ref):
    @pl.when(pl.program_id(1) == 0)
    def _():
        ss_ref[...] = jnp.zeros_like(ss_ref)

    g3 = gp_ref[0]
    x3 = gx_ref[0]
    mp = jnp.mean(g3, axis=1, keepdims=True)
    mx = jnp.mean(x3, axis=1, keepdims=True)
    cp = g3 - mp
    cx = x3 - mx
    tot = jnp.sum(cp * cp) + jnp.sum(cx * cx)
    ss_ref[...] += jnp.full((1, 1, 128), tot, jnp.float32)


def _stats(gp, gx):
    return pl.pallas_call(
        _stats_kernel,
        grid=(B, S // SB),
        in_specs=[
            pl.BlockSpec((1, SB, KNN, CP), lambda b, s: (b, s, 0, 0)),
            pl.BlockSpec((1, SB, KNN, CXP), lambda b, s: (b, s, 0, 0)),
        ],
        out_specs=pl.BlockSpec((1, 1, 128), lambda b, s: (b, 0, 0)),
        out_shape=jax.ShapeDtypeStruct((B, 1, 128), jnp.float32),
    )(gp, gx)


def _apply_kernel(gp_ref, gx_ref, ss_ref, np_ref, ap_ref, ax_ref,
                  bp_ref, bx_ref, out_ref):
    g3 = gp_ref[0]
    x3 = gx_ref[0]
    mp = jnp.mean(g3, axis=1, keepdims=True)
    mx = jnp.mean(x3, axis=1, keepdims=True)
    std = jnp.sqrt(ss_ref[0, 0, 0] / (NORM_N - 1))
    rstd = 1.0 / (std + 1e-5)
    outp = ap_ref[0][None, None, :] * ((g3 - mp) * rstd) + bp_ref[0][None, None, :]
    outx = ax_ref[0][None, None, :] * ((x3 - mx) * rstd) + bx_ref[0][None, None, :]
    npb = jnp.broadcast_to(np_ref[0][:, None, :], (SB, KNN, CP))
    out_ref[0, :, :, 0:CP] = outp
    out_ref[0, :, :, CP:CC] = outx[:, :, 0:CX]
    out_ref[0, :, :, CC:COUT] = npb


def _apply(gp, gx, ss, new_points, ap, ax, bp, bx):
    return pl.pallas_call(
        _apply_kernel,
        grid=(B, S // SB),
        in_specs=[
            pl.BlockSpec((1, SB, KNN, CP), lambda b, s: (b, s, 0, 0)),
            pl.BlockSpec((1, SB, KNN, CXP), lambda b, s: (b, s, 0, 0)),
            pl.BlockSpec((1, 1, 128), lambda b, s: (b, 0, 0)),
            pl.BlockSpec((1, SB, CP), lambda b, s: (b, s, 0)),
            pl.BlockSpec((1, CP), lambda b, s: (0, 0)),
            pl.BlockSpec((1, CXP), lambda b, s: (0, 0)),
            pl.BlockSpec((1, CP), lambda b, s: (0, 0)),
            pl.BlockSpec((1, CXP), lambda b, s: (0, 0)),
        ],
        out_specs=pl.BlockSpec((1, SB, KNN, COUT), lambda b, s: (b, s, 0, 0)),
        out_shape=jax.ShapeDtypeStruct((B, S, KNN, COUT), jnp.float32),
    )(gp, gx, ss, new_points, ap, ax, bp, bx)


SBK = 128
CD = 8


def _knn_kernel(q_ref, xt_ref, idx_ref, d_scr):
    q = q_ref[0]
    x = xt_ref[0]
    mm = jnp.dot(q.astype(jnp.bfloat16), x.astype(jnp.bfloat16),
                 preferred_element_type=jnp.float32)
    acc = -2.0 * mm
    qn = q[:, 0:1] * q[:, 0:1]
    for c in range(1, CX):
        qn = qn + q[:, c:c + 1] * q[:, c:c + 1]
    xn = x[0:1, :] * x[0:1, :]
    for c in range(1, CX):
        xn = xn + x[c:c + 1, :] * x[c:c + 1, :]
    acc = acc + qn
    acc = acc + xn
    d_scr[...] = acc
    cols = lax.broadcasted_iota(jnp.int32, (SBK, N), 1)
    j32 = lax.broadcasted_iota(jnp.int32, (SBK, KNN), 1)

    def body(j, idx_acc):
        dv = d_scr[...]
        m = jnp.min(dv, axis=1, keepdims=True)
        cand = jnp.where(dv == m, cols, jnp.int32(2 ** 30))
        a = jnp.min(cand, axis=1, keepdims=True)
        d_scr[...] = jnp.where(cand == a, jnp.float32(jnp.inf), dv)
        return jnp.where(j32 == j, a, idx_acc)

    idx_ref[0] = lax.fori_loop(0, KNN, body, jnp.zeros((SBK, KNN), jnp.int32))


def _knn(new_xyz_pad, xyz_t_pad):
    return pl.pallas_call(
        _knn_kernel,
        grid=(B, S // SBK),
        in_specs=[
            pl.BlockSpec((1, SBK, CD), lambda b, s: (b, s, 0)),
            pl.BlockSpec((1, CD, N), lambda b, s: (b, 0, 0)),
        ],
        out_specs=pl.BlockSpec((1, SBK, KNN), lambda b, s: (b, s, 0)),
        out_shape=jax.ShapeDtypeStruct((B, S, KNN), jnp.int32),
        scratch_shapes=[pltpu.VMEM((SBK, N), jnp.float32)],
    )(new_xyz_pad, xyz_t_pad)


def kernel(xyz, points, new_xyz, new_points, affine_alpha, affine_beta):
    q_pad = jnp.pad(new_xyz, ((0, 0), (0, 0), (0, CD - CX)))
    xyz_t = jnp.pad(jnp.swapaxes(xyz, 1, 2), ((0, 0), (0, CD - CX), (0, 0)))
    idx = _knn(q_pad, xyz_t)

    idx_flat = (idx + (jnp.arange(B, dtype=jnp.int32) * N)[:, None, None])
    idx_flat = idx_flat.reshape(-1).astype(jnp.int32)
    points_flat = points.reshape(B * N, CP)
    xyz_flat = jnp.pad(xyz, ((0, 0), (0, 0), (0, CXP - CX))).reshape(B * N, CXP)

    gp_flat, gx_flat = _sc_gather(points_flat, xyz_flat, idx_flat)
    gp = gp_flat.reshape(B, S, KNN, CP)
    gx = gx_flat.reshape(B, S, KNN, CXP)

    ss = _stats(gp, gx)
    a = affine_alpha.reshape(CC)
    bta = affine_beta.reshape(CC)
    ap = a[:CP].reshape(1, CP)
    ax = jnp.pad(a[CP:], (0, CXP - CX)).reshape(1, CXP)
    bp = bta[:CP].reshape(1, CP)
    bx = jnp.pad(bta[CP:], (0, CXP - CX)).reshape(1, CXP)
    out = _apply(gp, gx, ss, new_points, ap, ax, bp, bx)
    return (new_xyz, out)


# speedup vs baseline: 7.3871x; 1.4727x over previous
import functools

import jax
import jax.numpy as jnp
from jax import lax
from jax.experimental import pallas as pl
from jax.experimental.pallas import tpu as pltpu
from jax.experimental.pallas import tpu_sc as plsc

KNN = 32
B, N, S = 8, 8192, 512
CP = 256
CX = 5
CXP = 16
CC = CP + CX
COUT = CC + CP
NW = 32
GATHER_CHUNK = 128
SB = 128
NORM_N = S * KNN * CC


def _sc_gather(points_flat, xyz_flat, idx_flat):
    ni = idx_flat.shape[0]
    per_w = ni // NW
    n_chunks = per_w // GATHER_CHUNK
    mesh = plsc.VectorSubcoreMesh(core_axis_name="c", subcore_axis_name="s")

    @functools.partial(
        pl.kernel,
        mesh=mesh,
        compiler_params=pltpu.CompilerParams(use_tc_tiling_on_sc=False),
        out_type=(
            jax.ShapeDtypeStruct((ni, CP), jnp.float32),
            jax.ShapeDtypeStruct((ni, CXP), jnp.float32),
        ),
        scratch_types=[
            pltpu.VMEM((GATHER_CHUNK,), jnp.int32),
            pltpu.VMEM((GATHER_CHUNK, CP), jnp.float32),
            pltpu.VMEM((GATHER_CHUNK, CXP), jnp.float32),
            pltpu.SemaphoreType.DMA,
            pltpu.SemaphoreType.DMA,
        ],
    )
    def k(pts_hbm, xyz_hbm, idx_hbm, op_hbm, ox_hbm, idx_v, rp_v, rx_v, s1, s2):
        wid = lax.axis_index("s") * 2 + lax.axis_index("c")
        base = wid * per_w

        @pl.loop(0, n_chunks)
        def _(ci):
            off = base + ci * GATHER_CHUNK
            pltpu.sync_copy(idx_hbm.at[pl.ds(off, GATHER_CHUNK)], idx_v)
            c1 = pltpu.async_copy(pts_hbm.at[idx_v], rp_v, s1)
            c2 = pltpu.async_copy(xyz_hbm.at[idx_v], rx_v, s2)
            c1.wait()
            c2.wait()
            pltpu.sync_copy(rp_v, op_hbm.at[pl.ds(off, GATHER_CHUNK)])
            pltpu.sync_copy(rx_v, ox_hbm.at[pl.ds(off, GATHER_CHUNK)])

    return k(points_flat, xyz_flat, idx_flat)


def _stats_kernel(gp_ref, gx_ref, ss_ref):
    @pl.when(pl.program_id(1) == 0)
    def _():
        ss_ref[...] = jnp.zeros_like(ss_ref)

    g3 = gp_ref[0]
    x3 = gx_ref[0]
    mp = jnp.mean(g3, axis=1, keepdims=True)
    mx = jnp.mean(x3, axis=1, keepdims=True)
    cp = g3 - mp
    cx = x3 - mx
    tot = jnp.sum(cp * cp) + jnp.sum(cx * cx)
    ss_ref[...] += jnp.full((1, 1, 128), tot, jnp.float32)


def _stats(gp, gx):
    return pl.pallas_call(
        _stats_kernel,
        grid=(B, S // SB),
        in_specs=[
            pl.BlockSpec((1, SB, KNN, CP), lambda b, s: (b, s, 0, 0)),
            pl.BlockSpec((1, SB, KNN, CXP), lambda b, s: (b, s, 0, 0)),
        ],
        out_specs=pl.BlockSpec((1, 1, 128), lambda b, s: (b, 0, 0)),
        out_shape=jax.ShapeDtypeStruct((B, 1, 128), jnp.float32),
    )(gp, gx)


def _apply_kernel(gp_ref, gx_ref, ss_ref, np_ref, ap_ref, ax_ref,
                  bp_ref, bx_ref, out_ref):
    g3 = gp_ref[0]
    x3 = gx_ref[0]
    mp = jnp.mean(g3, axis=1, keepdims=True)
    mx = jnp.mean(x3, axis=1, keepdims=True)
    std = jnp.sqrt(ss_ref[0, 0, 0] / (NORM_N - 1))
    rstd = 1.0 / (std + 1e-5)
    outp = ap_ref[0][None, None, :] * ((g3 - mp) * rstd) + bp_ref[0][None, None, :]
    outx = ax_ref[0][None, None, :] * ((x3 - mx) * rstd) + bx_ref[0][None, None, :]
    npb = jnp.broadcast_to(np_ref[0][:, None, :], (SB, KNN, CP))
    out_ref[0, :, :, 0:CP] = outp
    out_ref[0, :, :, CP:CC] = outx[:, :, 0:CX]
    out_ref[0, :, :, CC:COUT] = npb


def _apply(gp, gx, ss, new_points, ap, ax, bp, bx):
    return pl.pallas_call(
        _apply_kernel,
        grid=(B, S // SB),
        in_specs=[
            pl.BlockSpec((1, SB, KNN, CP), lambda b, s: (b, s, 0, 0)),
            pl.BlockSpec((1, SB, KNN, CXP), lambda b, s: (b, s, 0, 0)),
            pl.BlockSpec((1, 1, 128), lambda b, s: (b, 0, 0)),
            pl.BlockSpec((1, SB, CP), lambda b, s: (b, s, 0)),
            pl.BlockSpec((1, CP), lambda b, s: (0, 0)),
            pl.BlockSpec((1, CXP), lambda b, s: (0, 0)),
            pl.BlockSpec((1, CP), lambda b, s: (0, 0)),
            pl.BlockSpec((1, CXP), lambda b, s: (0, 0)),
        ],
        out_specs=pl.BlockSpec((1, SB, KNN, COUT), lambda b, s: (b, s, 0, 0)),
        out_shape=jax.ShapeDtypeStruct((B, S, KNN, COUT), jnp.float32),
    )(gp, gx, ss, new_points, ap, ax, bp, bx)


SBK = 128
CD = 8
NG = N // 128
LV = 6


def _knn_kernel(q_ref, xt_ref, idx_ref, d_scr):
    q = q_ref[0]
    x = xt_ref[0]
    mm = jnp.dot(q.astype(jnp.bfloat16), x.astype(jnp.bfloat16),
                 preferred_element_type=jnp.float32)
    acc = -2.0 * mm
    qn = q[:, 0:1] * q[:, 0:1]
    for c in range(1, CX):
        qn = qn + q[:, c:c + 1] * q[:, c:c + 1]
    xn = x[0:1, :] * x[0:1, :]
    for c in range(1, CX):
        xn = xn + x[c:c + 1, :] * x[c:c + 1, :]
    acc = acc + qn
    acc = acc + xn
    d_scr[...] = acc
    j32 = lax.broadcasted_iota(jnp.int32, (SBK, KNN), 1)
    lane = lax.broadcasted_iota(jnp.int32, (SBK, 128), 1)
    INF = jnp.float32(jnp.inf)
    BIG = jnp.int32(2 ** 30)

    ms, gs = [], []
    for sg in range(SBK // 8):
        init = ([jnp.full((8, 128), INF, jnp.float32) for _ in range(LV)]
                + [jnp.zeros((8, 128), jnp.int32) for _ in range(LV)])

        def scan_g(g, carry, sg=sg):
            m = list(carry[:LV])
            a = list(carry[LV:])
            v = d_scr[pl.ds(sg * 8, 8), pl.ds(pl.multiple_of(g * 128, 128), 128)]
            c = [v < m[t] for t in range(LV)]
            for t in range(LV - 1, 0, -1):
                m[t] = jnp.where(c[t], jnp.where(c[t - 1], m[t - 1], v), m[t])
                a[t] = jnp.where(c[t], jnp.where(c[t - 1], a[t - 1], g), a[t])
            m[0] = jnp.where(c[0], v, m[0])
            a[0] = jnp.where(c[0], g, a[0])
            return tuple(m) + tuple(a)

        res = lax.fori_loop(0, NG, scan_g, tuple(init))
        ms.append(res[:LV])
        gs.append(res[LV:])
    hm = [jnp.concatenate([ms[sg][t] for sg in range(SBK // 8)], axis=0)
          for t in range(LV)]
    hg = [jnp.concatenate([gs[sg][t] for sg in range(SBK // 8)], axis=0)
          for t in range(LV)]

    def extract(j, carry):
        idx_acc, cnt = carry
        v = INF
        g = BIG
        for t in range(LV - 1, -1, -1):
            sel = cnt == t
            v = jnp.where(sel, hm[t], v)
            g = jnp.where(sel, hg[t], g)
        m = jnp.min(v, axis=1, keepdims=True)
        f = jnp.where(v == m, g * 128 + lane, BIG)
        n = jnp.min(f, axis=1, keepdims=True)
        idx_acc = jnp.where(j32 == j, n, idx_acc)
        cnt = cnt + jnp.where(lane == jnp.bitwise_and(n, 127), 1, 0)
        return idx_acc, cnt

    idx_fast, cnt = lax.fori_loop(
        0, KNN, extract,
        (jnp.zeros((SBK, KNN), jnp.int32), jnp.zeros((SBK, 128), jnp.int32)))
    suspect = jnp.max(cnt) >= LV

    def slow(_):
        cols = lax.broadcasted_iota(jnp.int32, (SBK, N), 1)

        def body(j, idx_acc):
            dv = d_scr[...]
            m = jnp.min(dv, axis=1, keepdims=True)
            cand = jnp.where(dv == m, cols, BIG)
            a = jnp.min(cand, axis=1, keepdims=True)
            d_scr[...] = jnp.where(cand == a, INF, dv)
            return jnp.where(j32 == j, a, idx_acc)

        return lax.fori_loop(0, KNN, body, jnp.zeros((SBK, KNN), jnp.int32))

    idx_ref[0] = lax.cond(suspect, slow, lambda _: idx_fast, None)


def _knn(new_xyz_pad, xyz_t_pad):
    return pl.pallas_call(
        _knn_kernel,
        grid=(B, S // SBK),
        in_specs=[
            pl.BlockSpec((1, SBK, CD), lambda b, s: (b, s, 0)),
            pl.BlockSpec((1, CD, N), lambda b, s: (b, 0, 0)),
        ],
        out_specs=pl.BlockSpec((1, SBK, KNN), lambda b, s: (b, s, 0)),
        out_shape=jax.ShapeDtypeStruct((B, S, KNN), jnp.int32),
        scratch_shapes=[pltpu.VMEM((SBK, N), jnp.float32)],
    )(new_xyz_pad, xyz_t_pad)


def kernel(xyz, points, new_xyz, new_points, affine_alpha, affine_beta):
    q_pad = jnp.pad(new_xyz, ((0, 0), (0, 0), (0, CD - CX)))
    xyz_t = jnp.pad(jnp.swapaxes(xyz, 1, 2), ((0, 0), (0, CD - CX), (0, 0)))
    idx = _knn(q_pad, xyz_t)

    idx_flat = (idx + (jnp.arange(B, dtype=jnp.int32) * N)[:, None, None])
    idx_flat = idx_flat.reshape(-1).astype(jnp.int32)
    points_flat = points.reshape(B * N, CP)
    xyz_flat = jnp.pad(xyz, ((0, 0), (0, 0), (0, CXP - CX))).reshape(B * N, CXP)

    gp_flat, gx_flat = _sc_gather(points_flat, xyz_flat, idx_flat)
    gp = gp_flat.reshape(B, S, KNN, CP)
    gx = gx_flat.reshape(B, S, KNN, CXP)

    ss = _stats(gp, gx)
    a = affine_alpha.reshape(CC)
    bta = affine_beta.reshape(CC)
    ap = a[:CP].reshape(1, CP)
    ax = jnp.pad(a[CP:], (0, CXP - CX)).reshape(1, CXP)
    bp = bta[:CP].reshape(1, CP)
    bx = jnp.pad(bta[CP:], (0, CXP - CX)).reshape(1, CXP)
    out = _apply(gp, gx, ss, new_points, ap, ax, bp, bx)
    return (new_xyz, out)


# speedup vs baseline: 8.4242x; 1.1404x over previous
import functools

import jax
import jax.numpy as jnp
from jax import lax
from jax.experimental import pallas as pl
from jax.experimental.pallas import tpu as pltpu
from jax.experimental.pallas import tpu_sc as plsc

KNN = 32
B, N, S = 8, 8192, 512
CP = 256
CX = 5
CXP = 16
CC = CP + CX
COUT = CC + CP
NW = 32
GATHER_CHUNK = 128
SB = 128
NORM_N = S * KNN * CC


def _sc_gather(points_flat, xyz_flat, idx_flat):
    ni = idx_flat.shape[0]
    per_w = ni // NW
    n_chunks = per_w // GATHER_CHUNK
    mesh = plsc.VectorSubcoreMesh(core_axis_name="c", subcore_axis_name="s")

    @functools.partial(
        pl.kernel,
        mesh=mesh,
        compiler_params=pltpu.CompilerParams(use_tc_tiling_on_sc=False),
        out_type=(
            jax.ShapeDtypeStruct((ni, CP), jnp.float32),
            jax.ShapeDtypeStruct((ni, CXP), jnp.float32),
        ),
        scratch_types=[
            pltpu.VMEM((GATHER_CHUNK,), jnp.int32),
            pltpu.VMEM((GATHER_CHUNK, CP), jnp.float32),
            pltpu.VMEM((GATHER_CHUNK, CXP), jnp.float32),
            pltpu.SemaphoreType.DMA,
            pltpu.SemaphoreType.DMA,
        ],
    )
    def k(pts_hbm, xyz_hbm, idx_hbm, op_hbm, ox_hbm, idx_v, rp_v, rx_v, s1, s2):
        wid = lax.axis_index("s") * 2 + lax.axis_index("c")
        base = wid * per_w

        @pl.loop(0, n_chunks)
        def _(ci):
            off = base + ci * GATHER_CHUNK
            pltpu.sync_copy(idx_hbm.at[pl.ds(off, GATHER_CHUNK)], idx_v)
            c1 = pltpu.async_copy(pts_hbm.at[idx_v], rp_v, s1)
            c2 = pltpu.async_copy(xyz_hbm.at[idx_v], rx_v, s2)
            c1.wait()
            c2.wait()
            pltpu.sync_copy(rp_v, op_hbm.at[pl.ds(off, GATHER_CHUNK)])
            pltpu.sync_copy(rx_v, ox_hbm.at[pl.ds(off, GATHER_CHUNK)])

    return k(points_flat, xyz_flat, idx_flat)


def _stats_kernel(gp_ref, gx_ref, ss_ref):
    @pl.when(pl.program_id(1) == 0)
    def _():
        ss_ref[...] = jnp.zeros_like(ss_ref)

    g3 = gp_ref[0]
    x3 = gx_ref[0]
    mp = jnp.mean(g3, axis=1, keepdims=True)
    mx = jnp.mean(x3, axis=1, keepdims=True)
    cp = g3 - mp
    cx = x3 - mx
    tot = jnp.sum(cp * cp) + jnp.sum(cx * cx)
    ss_ref[...] += jnp.full((1, 1, 128), tot, jnp.float32)


def _stats(gp, gx):
    return pl.pallas_call(
        _stats_kernel,
        grid=(B, S // SB),
        in_specs=[
            pl.BlockSpec((1, SB, KNN, CP), lambda b, s: (b, s, 0, 0)),
            pl.BlockSpec((1, SB, KNN, CXP), lambda b, s: (b, s, 0, 0)),
        ],
        out_specs=pl.BlockSpec((1, 1, 128), lambda b, s: (b, 0, 0)),
        out_shape=jax.ShapeDtypeStruct((B, 1, 128), jnp.float32),
    )(gp, gx)


def _apply_kernel(gp_ref, gx_ref, ss_ref, np_ref, ap_ref, ax_ref,
                  bp_ref, bx_ref, out_ref):
    g3 = gp_ref[0]
    x3 = gx_ref[0]
    mp = jnp.mean(g3, axis=1, keepdims=True)
    mx = jnp.mean(x3, axis=1, keepdims=True)
    std = jnp.sqrt(ss_ref[0, 0, 0] / (NORM_N - 1))
    rstd = 1.0 / (std + 1e-5)
    outp = ap_ref[0][None, None, :] * ((g3 - mp) * rstd) + bp_ref[0][None, None, :]
    outx = ax_ref[0][None, None, :] * ((x3 - mx) * rstd) + bx_ref[0][None, None, :]
    npb = jnp.broadcast_to(np_ref[0][:, None, :], (SB, KNN, CP))
    out_ref[0, :, :, 0:CP] = outp
    out_ref[0, :, :, CP:CC] = outx[:, :, 0:CX]
    out_ref[0, :, :, CC:COUT] = npb


def _apply(gp, gx, ss, new_points, ap, ax, bp, bx):
    return pl.pallas_call(
        _apply_kernel,
        grid=(B, S // SB),
        in_specs=[
            pl.BlockSpec((1, SB, KNN, CP), lambda b, s: (b, s, 0, 0)),
            pl.BlockSpec((1, SB, KNN, CXP), lambda b, s: (b, s, 0, 0)),
            pl.BlockSpec((1, 1, 128), lambda b, s: (b, 0, 0)),
            pl.BlockSpec((1, SB, CP), lambda b, s: (b, s, 0)),
            pl.BlockSpec((1, CP), lambda b, s: (0, 0)),
            pl.BlockSpec((1, CXP), lambda b, s: (0, 0)),
            pl.BlockSpec((1, CP), lambda b, s: (0, 0)),
            pl.BlockSpec((1, CXP), lambda b, s: (0, 0)),
        ],
        out_specs=pl.BlockSpec((1, SB, KNN, COUT), lambda b, s: (b, s, 0, 0)),
        out_shape=jax.ShapeDtypeStruct((B, S, KNN, COUT), jnp.float32),
    )(gp, gx, ss, new_points, ap, ax, bp, bx)


SBK = 128
CD = 8
NG = N // 128
LV = 6


def _knn_kernel(q_ref, xt_ref, idx_ref, d_scr):
    q = q_ref[0]
    x = xt_ref[0]
    mm = jnp.dot(q.astype(jnp.bfloat16), x.astype(jnp.bfloat16),
                 preferred_element_type=jnp.float32)
    acc = -2.0 * mm
    qn = q[:, 0:1] * q[:, 0:1]
    for c in range(1, CX):
        qn = qn + q[:, c:c + 1] * q[:, c:c + 1]
    xn = x[0:1, :] * x[0:1, :]
    for c in range(1, CX):
        xn = xn + x[c:c + 1, :] * x[c:c + 1, :]
    acc = acc + qn
    acc = acc + xn
    d_scr[...] = acc
    j32 = lax.broadcasted_iota(jnp.int32, (SBK, KNN), 1)
    lane = lax.broadcasted_iota(jnp.int32, (SBK, 128), 1)
    INF = jnp.float32(jnp.inf)
    BIG = jnp.int32(2 ** 30)

    ms, gs = [], []
    for sg in range(SBK // 8):
        init = ([jnp.full((8, 128), INF, jnp.float32) for _ in range(LV)]
                + [jnp.zeros((8, 128), jnp.int32) for _ in range(LV)])

        def scan_g(g, carry, sg=sg):
            m = list(carry[:LV])
            a = list(carry[LV:])
            v = d_scr[pl.ds(sg * 8, 8), pl.ds(pl.multiple_of(g * 128, 128), 128)]
            c = [v < m[t] for t in range(LV)]
            for t in range(LV - 1, 0, -1):
                m[t] = jnp.where(c[t], jnp.where(c[t - 1], m[t - 1], v), m[t])
                a[t] = jnp.where(c[t], jnp.where(c[t - 1], a[t - 1], g), a[t])
            m[0] = jnp.where(c[0], v, m[0])
            a[0] = jnp.where(c[0], g, a[0])
            return tuple(m) + tuple(a)

        res = lax.fori_loop(0, NG, scan_g, tuple(init), unroll=8)
        ms.append(res[:LV])
        gs.append(res[LV:])
    hm = [jnp.concatenate([ms[sg][t] for sg in range(SBK // 8)], axis=0)
          for t in range(LV)]
    hg = [jnp.concatenate([gs[sg][t] for sg in range(SBK // 8)], axis=0)
          for t in range(LV)]

    def extract(j, carry):
        idx_acc, cnt = carry
        v = INF
        g = BIG
        for t in range(LV - 1, -1, -1):
            sel = cnt == t
            v = jnp.where(sel, hm[t], v)
            g = jnp.where(sel, hg[t], g)
        m = jnp.min(v, axis=1, keepdims=True)
        f = jnp.where(v == m, g * 128 + lane, BIG)
        n = jnp.min(f, axis=1, keepdims=True)
        idx_acc = jnp.where(j32 == j, n, idx_acc)
        cnt = cnt + jnp.where(lane == jnp.bitwise_and(n, 127), 1, 0)
        return idx_acc, cnt

    idx_fast, cnt = lax.fori_loop(
        0, KNN, extract,
        (jnp.zeros((SBK, KNN), jnp.int32), jnp.zeros((SBK, 128), jnp.int32)),
        unroll=4)
    suspect = jnp.max(cnt) >= LV

    def slow(_):
        cols = lax.broadcasted_iota(jnp.int32, (SBK, N), 1)

        def body(j, idx_acc):
            dv = d_scr[...]
            m = jnp.min(dv, axis=1, keepdims=True)
            cand = jnp.where(dv == m, cols, BIG)
            a = jnp.min(cand, axis=1, keepdims=True)
            d_scr[...] = jnp.where(cand == a, INF, dv)
            return jnp.where(j32 == j, a, idx_acc)

        return lax.fori_loop(0, KNN, body, jnp.zeros((SBK, KNN), jnp.int32))

    idx_ref[0] = lax.cond(suspect, slow, lambda _: idx_fast, None)


def _knn(new_xyz_pad, xyz_t_pad):
    return pl.pallas_call(
        _knn_kernel,
        grid=(B, S // SBK),
        in_specs=[
            pl.BlockSpec((1, SBK, CD), lambda b, s: (b, s, 0)),
            pl.BlockSpec((1, CD, N), lambda b, s: (b, 0, 0)),
        ],
        out_specs=pl.BlockSpec((1, SBK, KNN), lambda b, s: (b, s, 0)),
        out_shape=jax.ShapeDtypeStruct((B, S, KNN), jnp.int32),
        scratch_shapes=[pltpu.VMEM((SBK, N), jnp.float32)],
    )(new_xyz_pad, xyz_t_pad)


def kernel(xyz, points, new_xyz, new_points, affine_alpha, affine_beta):
    q_pad = jnp.pad(new_xyz, ((0, 0), (0, 0), (0, CD - CX)))
    xyz_t = jnp.pad(jnp.swapaxes(xyz, 1, 2), ((0, 0), (0, CD - CX), (0, 0)))
    idx = _knn(q_pad, xyz_t)

    idx_flat = (idx + (jnp.arange(B, dtype=jnp.int32) * N)[:, None, None])
    idx_flat = idx_flat.reshape(-1).astype(jnp.int32)
    points_flat = points.reshape(B * N, CP)
    xyz_flat = jnp.pad(xyz, ((0, 0), (0, 0), (0, CXP - CX))).reshape(B * N, CXP)

    gp_flat, gx_flat = _sc_gather(points_flat, xyz_flat, idx_flat)
    gp = gp_flat.reshape(B, S, KNN, CP)
    gx = gx_flat.reshape(B, S, KNN, CXP)

    ss = _stats(gp, gx)
    a = affine_alpha.reshape(CC)
    bta = affine_beta.reshape(CC)
    ap = a[:CP].reshape(1, CP)
    ax = jnp.pad(a[CP:], (0, CXP - CX)).reshape(1, CXP)
    bp = bta[:CP].reshape(1, CP)
    bx = jnp.pad(bta[CP:], (0, CXP - CX)).reshape(1, CXP)
    out = _apply(gp, gx, ss, new_points, ap, ax, bp, bx)
    return (new_xyz, out)
